# Initial kernel scaffold; baseline (speedup 1.0000x reference)
import jax
import jax.numpy as jnp
from jax.experimental import pallas as pl


def kernel(X, negative_sample_indices, W):
    raise NotImplementedError("write your pallas kernel here")


# speedup vs baseline: 5.4878x; 5.4878x over previous
import functools

import jax
import jax.numpy as jnp
from jax import lax
from jax.experimental import pallas as pl
from jax.experimental.pallas import tpu as pltpu
from jax.experimental.pallas import tpu_sc as plsc

_N = 16384
_E = 1
_C = 5
_SL = 20
_D = 64
_K = _E + _C + _SL
_NW = 32
_ROWS_PER_W = _N // _NW
_CH = 64
_NCHUNK = _ROWS_PER_W // _CH
_GPC = _CH * _K // 128
_L = 16


def _make_kernel():
    mesh = plsc.VectorSubcoreMesh(core_axis_name="c", subcore_axis_name="s")

    @functools.partial(
        pl.kernel,
        out_type=jax.ShapeDtypeStruct((_N, 1 + _SL), jnp.float32),
        mesh=mesh,
        compiler_params=pltpu.CompilerParams(
            needs_layout_passes=False, use_tc_tiling_on_sc=False
        ),
        scratch_types=[
            pltpu.VMEM((_ROWS_PER_W * _K // 128, 128), jnp.int32),
            pltpu.VMEM((_CH * _K, _D), jnp.float32),
            pltpu.VMEM((_CH, 1 + _SL), jnp.float32),
            pltpu.SemaphoreType.DMA,
        ],
    )
    def body(w_hbm, idx_hbm, out_hbm, idx_v, rows_v, out_v, sem):
        wid = lax.axis_index("s") * 2 + lax.axis_index("c")
        blocks_per_w = _ROWS_PER_W * _K // 128

        pltpu.sync_copy(idx_hbm.at[pl.ds(wid * blocks_per_w, blocks_per_w)], idx_v)

        for c in range(_NCHUNK):
            row_base = wid * _ROWS_PER_W + c * _CH

            copies = []
            for j in range(_GPC):
                copies.append(
                    pltpu.async_copy(
                        w_hbm.at[idx_v.at[c * _GPC + j]],
                        rows_v.at[pl.ds(j * 128, 128)],
                        sem,
                    )
                )
            for cp in copies:
                cp.wait()

            for g in range(_CH // _L):
                row_ids = jax.lax.broadcasted_iota(jnp.int32, (_L,), 0) + g * _L
                slot0 = row_ids * _K

                def dbody(d, accs, slot0=slot0):
                    col = jnp.full((_L,), 0, jnp.int32) + d
                    bc = plsc.load_gather(rows_v, [slot0 + 1, col])
                    for j in range(2, _C + 1):
                        bc = bc + plsc.load_gather(rows_v, [slot0 + j, col])
                    bc = bc * jnp.float32(1.0 / _C)
                    be = plsc.load_gather(rows_v, [slot0, col])
                    news = [accs[0] + be * bc]
                    for s in range(_SL):
                        ws = plsc.load_gather(rows_v, [slot0 + _E + _C + s, col])
                        news.append(accs[1 + s] + bc * ws)
                    return tuple(news)

                init = tuple(jnp.zeros((_L,), jnp.float32) for _ in range(1 + _SL))
                accs = lax.fori_loop(0, _D, dbody, init)

                for s in range(1 + _SL):
                    plsc.store_scatter(
                        out_v,
                        [row_ids, jnp.full((_L,), s, jnp.int32)],
                        accs[s],
                    )

            pltpu.sync_copy(out_v, out_hbm.at[pl.ds(row_base, _CH)])

    return body


_kernel_call = _make_kernel()


def kernel(X, negative_sample_indices, W):
    idx = jnp.concatenate([X, negative_sample_indices], axis=1)
    idx = idx.reshape(_N * _K // 128, 128)
    return _kernel_call(W, idx)
